# baseline (speedup 1.0000x reference)
import functools

import jax
import jax.numpy as jnp
from jax import lax
from jax.experimental import pallas as pl
from jax.experimental.pallas import tpu as pltpu
from jax.experimental.pallas import tpu_sc as plsc

B = 16384
N_CAT = 26
N_CONT = 10
VOCAB = 100000
H = 32
NF = N_CAT + N_CONT

NC = 2
NS = 16
NW = NC * NS
BPW = B // NW
CB = 64
NCHUNK = BPW // CB
RCAT = CB * N_CAT
RCONT = CB * N_CONT
GCAT = RCAT // 128
GCONT = RCONT // 128
LANES = 16


def _body(cat_hbm, cont_hbm, tab_hbm, w_hbm, out_hbm,
          w_v, featoff_v, dpc_v, dpn_v,
          catstage_v, srcidx_v, dstc_v, dstn_v,
          catrows_v, controws_v, cont_v,
          gsem, ssem):
    cid = lax.axis_index("c")
    sid = lax.axis_index("s")
    wid = sid * NC + cid

    pltpu.sync_copy(w_hbm, w_v)

    def pat_cat(t, _):
        r = t // 8
        k = t - r * 8
        e = r * 128 + k * 16 + lax.iota(jnp.int32, 16)
        brow = (e * 2521) >> 16
        f = e - brow * N_CAT
        featoff_v[r, pl.ds(k * 16, LANES)] = f * VOCAB
        dpc_v[r, pl.ds(k * 16, LANES)] = brow * NF + f
        return 0

    lax.fori_loop(0, GCAT * 8, pat_cat, 0)

    def pat_cont(t, _):
        r = t // 8
        k = t - r * 8
        e = r * 128 + k * 16 + lax.iota(jnp.int32, 16)
        brow = (e * 6554) >> 16
        j = e - brow * N_CONT
        dpn_v[r, pl.ds(k * 16, LANES)] = brow * NF + N_CAT + j
        return 0

    lax.fori_loop(0, GCONT * 8, pat_cont, 0)

    def chunk(c, _):
        base = wid * BPW + c * CB
        pltpu.sync_copy(cat_hbm.at[pl.ds(base * N_CAT, RCAT)], catstage_v)
        pltpu.sync_copy(cont_hbm.at[pl.ds(base * N_CONT, RCONT)], cont_v)

        def idx_cat(t, _):
            r = t // 8
            k = t - r * 8
            s = pl.ds(k * 16, LANES)
            srcidx_v[r, s] = catstage_v[pl.ds(t * 16, LANES)] + featoff_v[r, s]
            dstc_v[r, s] = dpc_v[r, s] + base * NF
            return 0

        lax.fori_loop(0, GCAT * 8, idx_cat, 0)

        def idx_cont(t, _):
            r = t // 8
            k = t - r * 8
            s = pl.ds(k * 16, LANES)
            dstn_v[r, s] = dpn_v[r, s] + base * NF
            return 0

        lax.fori_loop(0, GCONT * 8, idx_cont, 0)

        gathers = []
        for j in range(GCAT):
            gathers.append(pltpu.async_copy(
                tab_hbm.at[srcidx_v.at[j]],
                catrows_v.at[pl.ds(j * 128, 128)],
                gsem))

        def cont_body(v, _):
            bj = v // 2
            half = v - bj * 2
            cval = plsc.load_gather(
                cont_v, [jnp.full((LANES,), bj, jnp.int32)])
            wv = w_v[pl.ds((v % (2 * N_CONT)) * 16, LANES)]
            controws_v[bj, pl.ds(half * 16, LANES)] = cval * wv
            return 0

        lax.fori_loop(0, RCONT * 2, cont_body, 0)

        for g in gathers:
            g.wait()

        scatters = []
        for j in range(GCAT):
            scatters.append(pltpu.async_copy(
                catrows_v.at[pl.ds(j * 128, 128)],
                out_hbm.at[dstc_v.at[j]],
                ssem))
        for j in range(GCONT):
            scatters.append(pltpu.async_copy(
                controws_v.at[pl.ds(j * 128, 128)],
                out_hbm.at[dstn_v.at[j]],
                ssem))
        for s in scatters:
            s.wait()
        return 0

    lax.fori_loop(0, NCHUNK, chunk, 0)


@jax.jit
def _run(cat2d, cont2d, tab2d, wflat):
    mesh = plsc.VectorSubcoreMesh(
        core_axis_name="c", subcore_axis_name="s",
        num_cores=NC, num_subcores=NS)
    f = pl.kernel(
        _body,
        out_type=jax.ShapeDtypeStruct((B * NF, H), jnp.float32),
        mesh=mesh,
        compiler_params=pltpu.CompilerParams(
            needs_layout_passes=False, use_tc_tiling_on_sc=False),
        scratch_types=[
            pltpu.VMEM((N_CONT * H,), jnp.float32),
            pltpu.VMEM((GCAT, 128), jnp.int32),
            pltpu.VMEM((GCAT, 128), jnp.int32),
            pltpu.VMEM((GCONT, 128), jnp.int32),
            pltpu.VMEM((RCAT,), jnp.int32),
            pltpu.VMEM((GCAT, 128), jnp.int32),
            pltpu.VMEM((GCAT, 128), jnp.int32),
            pltpu.VMEM((GCONT, 128), jnp.int32),
            pltpu.VMEM((RCAT, H), jnp.float32),
            pltpu.VMEM((RCONT, H), jnp.float32),
            pltpu.VMEM((RCONT,), jnp.float32),
            pltpu.SemaphoreType.DMA,
            pltpu.SemaphoreType.DMA,
        ],
    )
    return f(cat2d, cont2d, tab2d, wflat)


def kernel(categorical, continuous, tables, W):
    cat1d = categorical.astype(jnp.int32).reshape(B * N_CAT)
    cont1d = continuous.reshape(B * N_CONT)
    tab2d = tables.reshape(N_CAT * VOCAB, H)
    wflat = W.reshape(N_CONT * H)
    out = _run(cat1d, cont1d, tab2d, wflat)
    return out.reshape(B, NF, H)


# speedup vs baseline: 1.0191x; 1.0191x over previous
import jax
import jax.numpy as jnp
from jax import lax
from jax.experimental import pallas as pl
from jax.experimental.pallas import tpu as pltpu
from jax.experimental.pallas import tpu_sc as plsc

B = 16384
N_CAT = 26
N_CONT = 10
VOCAB = 100000
H = 32
NF = N_CAT + N_CONT

NC = 2
NS = 16
NW = NC * NS
BPW = B // NW
NBT = BPW // 128
G = 2
NG = NF // G
LANES = 16


def _body(cat_hbm, cont_hbm, tab_hbm, w_hbm, out_hbm,
          catstage_v, contstage_v, w_v,
          srcidx0, srcidx1, srcidx2,
          grows0, grows1, grows2, stage0, stage1,
          gsem0, gsem1, gsem2, osem0, osem1):
    cid = lax.axis_index("c")
    sid = lax.axis_index("s")
    wid = sid * NC + cid
    base = wid * BPW

    pltpu.sync_copy(w_hbm, w_v)
    pltpu.sync_copy(cat_hbm.at[:, pl.ds(base, BPW)], catstage_v)
    pltpu.sync_copy(cont_hbm.at[:, pl.ds(base, BPW)], contstage_v)

    srcidx = (srcidx0, srcidx1, srcidx2)
    grows = (grows0, grows1, grows2)
    stage = (stage0, stage1)
    gsem = (gsem0, gsem1, gsem2)
    osem = (osem0, osem1)

    def ncat_of(g):
        return max(0, min(G, N_CAT - g * G))

    def chunk(bt, _):
        btg = wid * NBT + bt
        gdescs = {0: [], 1: [], 2: []}
        odescs = {0: [], 1: []}

        def fire(g):
            buf = g % 3
            ncat = ncat_of(g)
            if not ncat:
                return

            def idx_body(t, _, g=g, buf=buf):
                fl = t // 8
                k = t - fl * 8
                f = g * G + fl
                s = pl.ds(k * 16, LANES)
                v = catstage_v[f, pl.ds(bt * 128 + k * 16, LANES)]
                srcidx[buf][fl, s] = v + f * VOCAB
                return 0

            lax.fori_loop(0, ncat * 8, idx_body, 0)
            for fl in range(ncat):
                gdescs[buf].append(pltpu.async_copy(
                    tab_hbm.at[srcidx[buf].at[fl]],
                    grows[buf].at[pl.ds(fl * 128, 128)],
                    gsem[buf]))

        def process(g):
            buf = g % 3
            sbuf = g % 2
            ncat = ncat_of(g)
            for d in odescs[sbuf]:
                d.wait()
            odescs[sbuf] = []
            for d in gdescs[buf]:
                d.wait()
            gdescs[buf] = []

            for fl in range(ncat, G):
                j = g * G + fl - N_CAT

                def cont_body(t, _, fl=fl, j=j, sbuf=sbuf):
                    h = t
                    wv = plsc.load_gather(
                        w_v, [jnp.full((LANES,), j, jnp.int32),
                              jnp.full((LANES,), h, jnp.int32)])
                    for bq in range(8):
                        c = contstage_v[j, pl.ds(bt * 128 + bq * 16, LANES)]
                        stage[sbuf][fl * H + h, pl.ds(bq * 16, LANES)] = c * wv
                    return 0

                lax.fori_loop(0, H, cont_body, 0)

            if ncat:
                def tr_body(t, _, buf=buf, sbuf=sbuf):
                    fl = t // 8
                    lg = t - fl * 8
                    rows = fl * 128 + lg * 16 + lax.iota(jnp.int32, 16)
                    for h in range(H):
                        vals = plsc.load_gather(
                            grows[buf], [rows, jnp.full((LANES,), h, jnp.int32)])
                        stage[sbuf][fl * H + h, pl.ds(lg * 16, LANES)] = vals
                    return 0

                lax.fori_loop(0, ncat * 8, tr_body, 0)

            for fl in range(G):
                f = g * G + fl
                for sg in range(4):
                    odescs[sbuf].append(pltpu.async_copy(
                        stage[sbuf].at[pl.ds(fl * H + sg * 8, 8)],
                        out_hbm.at[f, sg, btg],
                        osem[sbuf]))

        fire(0)
        fire(1)
        for g in range(2, NG):
            fire(g)
            process(g - 2)
        process(NG - 2)
        process(NG - 1)
        for p in (0, 1):
            for d in odescs[p]:
                d.wait()
        return 0

    lax.fori_loop(0, NBT, chunk, 0)


@jax.jit
def _run(cat_t, cont_t, tabrm, W):
    mesh = plsc.VectorSubcoreMesh(
        core_axis_name="c", subcore_axis_name="s",
        num_cores=NC, num_subcores=NS)
    f = pl.kernel(
        _body,
        out_type=jax.ShapeDtypeStruct((NF, 4, 128, 8, 128), jnp.float32),
        mesh=mesh,
        compiler_params=pltpu.CompilerParams(
            needs_layout_passes=False, use_tc_tiling_on_sc=False),
        scratch_types=[
            pltpu.VMEM((N_CAT, BPW), jnp.int32),
            pltpu.VMEM((N_CONT, BPW), jnp.float32),
            pltpu.VMEM((N_CONT, H), jnp.float32),
            pltpu.VMEM((G, 128), jnp.int32),
            pltpu.VMEM((G, 128), jnp.int32),
            pltpu.VMEM((G, 128), jnp.int32),
            pltpu.VMEM((G * 128, H), jnp.float32),
            pltpu.VMEM((G * 128, H), jnp.float32),
            pltpu.VMEM((G * 128, H), jnp.float32),
            pltpu.VMEM((G * H, 128), jnp.float32),
            pltpu.VMEM((G * H, 128), jnp.float32),
            pltpu.SemaphoreType.DMA,
            pltpu.SemaphoreType.DMA,
            pltpu.SemaphoreType.DMA,
            pltpu.SemaphoreType.DMA,
            pltpu.SemaphoreType.DMA,
        ],
    )
    return f(cat_t, cont_t, tabrm, W)


def kernel(categorical, continuous, tables, W):
    cat_t = categorical.astype(jnp.int32).T
    cont_t = continuous.T
    tabrm = tables.reshape(N_CAT * VOCAB, H)
    out = _run(cat_t, cont_t, tabrm, W)
    return out.transpose(2, 4, 0, 1, 3).reshape(B, NF, H)
